# Initial kernel scaffold; baseline (speedup 1.0000x reference)
import functools

import jax
import jax.numpy as jnp
from jax import lax
from jax.experimental import pallas as pl
from jax.experimental.pallas import tpu as pltpu
from jax.experimental.pallas import tpu_sc as plsc

NC = 2
NS = 16
LANES = 16


@functools.lru_cache(maxsize=None)
def _make_sc_scatter(N, E, D):
    NW = NC * NS
    assert E % NW == 0, E
    e_per_w = E // NW
    CH = 80
    assert e_per_w % CH == 0, e_per_w
    n_ch = e_per_w // CH
    ZCH = CH
    n_pad = -(-N // (NS * ZCH)) * (NS * ZCH)
    z_per_tile = n_pad // NS
    n_z = z_per_tile // ZCH
    assert N % NS == 0, N
    rows_out = N // NS

    mesh = plsc.VectorSubcoreMesh(core_axis_name="c", subcore_axis_name="s")

    @functools.partial(
        pl.kernel,
        out_type=jax.ShapeDtypeStruct((NC, N, D), jnp.float32),
        mesh=mesh,
        scratch_types=[
            pltpu.MemoryRef((n_pad, D), jnp.float32, memory_space=pltpu.MemorySpace.VMEM_SHARED),
            pltpu.MemoryRef((CH, D), jnp.float32, memory_space=pltpu.MemorySpace.VMEM),
            pltpu.MemoryRef((CH,), jnp.int32, memory_space=pltpu.MemorySpace.VMEM),
            pltpu.MemoryRef((CH,), jnp.int32, memory_space=pltpu.MemorySpace.VMEM),
        ],
    )
    def sc_scatter(e_hbm, src_hbm, dst_hbm, out_hbm, acc, e_v, si_v, di_v):
        c = lax.axis_index("c")
        s = lax.axis_index("s")
        wid = c * NS + s

        def zrow(i, carry):
            for j in range(D // LANES):
                e_v[i, pl.ds(j * LANES, LANES)] = jnp.zeros((LANES,), jnp.float32)
            return carry

        lax.fori_loop(0, CH, zrow, 0)

        def zcopy(i, carry):
            pltpu.sync_copy(e_v, acc.at[pl.ds(s * z_per_tile + i * ZCH, ZCH)])
            return carry

        lax.fori_loop(0, n_z, zcopy, 0)
        plsc.subcore_barrier()

        base0 = wid * e_per_w

        def step(i, carry):
            base = base0 + i * CH
            pltpu.sync_copy(src_hbm.at[pl.ds(base, CH)], si_v)
            pltpu.sync_copy(dst_hbm.at[pl.ds(base, CH)], di_v)
            pltpu.sync_copy(e_hbm.at[pl.ds(base, CH)], e_v)
            pltpu.sync_copy(e_v, acc.at[si_v], add=True)
            pltpu.sync_copy(e_v, acc.at[di_v], add=True)
            return carry

        lax.fori_loop(0, n_ch, step, 0)
        plsc.subcore_barrier()

        pltpu.sync_copy(
            acc.at[pl.ds(s * rows_out, rows_out)],
            out_hbm.at[c, pl.ds(s * rows_out, rows_out)],
        )

    return sc_scatter


@functools.lru_cache(maxsize=None)
def _make_tc_matmul(N, D):
    BN = 2000
    assert N % BN == 0

    def mm(v_ref, p0_ref, p1_ref, w0_ref, w1_ref, b_ref, o_ref):
        v1 = p0_ref[...] + p1_ref[...]
        o_ref[...] = (
            jnp.dot(v_ref[...], w0_ref[...], preferred_element_type=jnp.float32)
            + jnp.dot(v1, w1_ref[...], preferred_element_type=jnp.float32)
            + b_ref[...]
        )

    return pl.pallas_call(
        mm,
        grid=(N // BN,),
        in_specs=[
            pl.BlockSpec((BN, D), lambda i: (i, 0)),
            pl.BlockSpec((BN, D), lambda i: (i, 0)),
            pl.BlockSpec((BN, D), lambda i: (i, 0)),
            pl.BlockSpec((D, D), lambda i: (0, 0)),
            pl.BlockSpec((D, D), lambda i: (0, 0)),
            pl.BlockSpec((1, D), lambda i: (0, 0)),
        ],
        out_specs=pl.BlockSpec((BN, D), lambda i: (i, 0)),
        out_shape=jax.ShapeDtypeStruct((N, D), jnp.float32),
    )


def kernel(e, v, edges, W, b):
    E, D = e.shape
    N = v.shape[0]
    src = edges[:, 0]
    dst = edges[:, 1]
    parts = _make_sc_scatter(N, E, D)(e, src, dst)
    mm = _make_tc_matmul(N, D)
    return mm(v, parts[0], parts[1], W[:D], W[D:], b.reshape(1, D))


# speedup vs baseline: 4.5419x; 4.5419x over previous
import functools

import jax
import jax.numpy as jnp
from jax import lax
from jax.experimental import pallas as pl
from jax.experimental.pallas import tpu as pltpu
from jax.experimental.pallas import tpu_sc as plsc

NC = 2
NS = 16
LANES = 16


@functools.lru_cache(maxsize=None)
def _make_sc_scatter(N, E, D):
    NW = NC * NS
    assert E % NW == 0, E
    e_per_w = E // NW
    CH = 80
    assert e_per_w % CH == 0, e_per_w
    n_ch = e_per_w // CH
    ZCH = CH
    n_pad = -(-N // (NS * ZCH)) * (NS * ZCH)
    z_per_tile = n_pad // NS
    n_z = z_per_tile // ZCH

    mesh = plsc.VectorSubcoreMesh(core_axis_name="c", subcore_axis_name="s")

    @functools.partial(
        pl.kernel,
        out_type=(
            jax.ShapeDtypeStruct((n_pad, D), jnp.float32),
            jax.ShapeDtypeStruct((n_pad, D), jnp.float32),
        ),
        mesh=mesh,
        scratch_types=[
            pltpu.VMEM_SHARED((n_pad, D), jnp.float32),
            pltpu.VMEM((CH, D), jnp.float32),
            pltpu.VMEM((CH,), jnp.int32),
            pltpu.VMEM((CH,), jnp.int32),
        ],
    )
    def sc_scatter(e_hbm, src_hbm, dst_hbm, out0_hbm, out1_hbm, acc, e_v, si_v, di_v):
        c = lax.axis_index("c")
        s = lax.axis_index("s")
        wid = c * NS + s

        def zrow(i, carry):
            for j in range(D // LANES):
                e_v[i, pl.ds(j * LANES, LANES)] = jnp.zeros((LANES,), jnp.float32)
            return carry

        lax.fori_loop(0, CH, zrow, 0)

        def zcopy(i, carry):
            pltpu.sync_copy(e_v, acc.at[pl.ds(s * z_per_tile + i * ZCH, ZCH)])
            return carry

        lax.fori_loop(0, n_z, zcopy, 0)
        plsc.subcore_barrier()

        base0 = wid * e_per_w

        def step(i, carry):
            base = base0 + i * CH
            pltpu.sync_copy(src_hbm.at[pl.ds(base, CH)], si_v)
            pltpu.sync_copy(dst_hbm.at[pl.ds(base, CH)], di_v)
            pltpu.sync_copy(e_hbm.at[pl.ds(base, CH)], e_v)
            pltpu.sync_copy(e_v, acc.at[si_v], add=True)
            pltpu.sync_copy(e_v, acc.at[di_v], add=True)
            return carry

        lax.fori_loop(0, n_ch, step, 0)
        plsc.subcore_barrier()

        row_slice = pl.ds(s * z_per_tile, z_per_tile)

        @pl.when(c == 0)
        def _():
            pltpu.sync_copy(acc.at[row_slice], out0_hbm.at[row_slice])

        @pl.when(c == 1)
        def _():
            pltpu.sync_copy(acc.at[row_slice], out1_hbm.at[row_slice])

    return sc_scatter


@functools.lru_cache(maxsize=None)
def _make_tc_matmul(N, D):
    BN = 2000
    assert N % BN == 0

    def mm(v_ref, p0_ref, p1_ref, w0_ref, w1_ref, b_ref, o_ref):
        v1 = p0_ref[...] + p1_ref[...]
        o_ref[...] = (
            jnp.dot(v_ref[...], w0_ref[...], preferred_element_type=jnp.float32)
            + jnp.dot(v1, w1_ref[...], preferred_element_type=jnp.float32)
            + b_ref[...]
        )

    return pl.pallas_call(
        mm,
        grid=(N // BN,),
        in_specs=[
            pl.BlockSpec((BN, D), lambda i: (i, 0)),
            pl.BlockSpec((BN, D), lambda i: (i, 0)),
            pl.BlockSpec((BN, D), lambda i: (i, 0)),
            pl.BlockSpec((D, D), lambda i: (0, 0)),
            pl.BlockSpec((D, D), lambda i: (0, 0)),
            pl.BlockSpec((1, D), lambda i: (0, 0)),
        ],
        out_specs=pl.BlockSpec((BN, D), lambda i: (i, 0)),
        out_shape=jax.ShapeDtypeStruct((N, D), jnp.float32),
    )


def kernel(e, v, edges, W, b):
    E, D = e.shape
    N = v.shape[0]
    src = edges[:, 0]
    dst = edges[:, 1]
    p0, p1 = _make_sc_scatter(N, E, D)(e, src, dst)
    mm = _make_tc_matmul(N, D)
    return mm(v, p0, p1, W[:D], W[D:], b.reshape(1, D))


# speedup vs baseline: 9.2659x; 2.0401x over previous
import functools

import jax
import jax.numpy as jnp
from jax import lax
from jax.experimental import pallas as pl
from jax.experimental.pallas import tpu as pltpu
from jax.experimental.pallas import tpu_sc as plsc

NC = 2
NS = 16
LANES = 16


@functools.lru_cache(maxsize=None)
def _make_sc_scatter(N, E, D):
    NW = NC * NS
    assert E % NW == 0, E
    e_per_w = E // NW
    CH = 80
    assert e_per_w % CH == 0, e_per_w
    n_ch = e_per_w // CH
    ZCH = CH
    n_pad = -(-N // (NS * ZCH)) * (NS * ZCH)
    z_per_tile = n_pad // NS
    n_z = z_per_tile // ZCH

    mesh = plsc.VectorSubcoreMesh(core_axis_name="c", subcore_axis_name="s")

    @functools.partial(
        pl.kernel,
        out_type=(
            jax.ShapeDtypeStruct((n_pad, D), jnp.float32),
            jax.ShapeDtypeStruct((n_pad, D), jnp.float32),
        ),
        mesh=mesh,
        scratch_types=[
            pltpu.VMEM_SHARED((n_pad, D), jnp.float32),
            pltpu.VMEM((CH, D), jnp.float32),
            pltpu.VMEM((CH, D), jnp.float32),
            pltpu.VMEM((CH,), jnp.int32),
            pltpu.VMEM((CH,), jnp.int32),
            pltpu.VMEM((CH,), jnp.int32),
            pltpu.VMEM((CH,), jnp.int32),
            pltpu.SemaphoreType.DMA,
            pltpu.SemaphoreType.DMA,
        ],
    )
    def sc_scatter(e_hbm, src_hbm, dst_hbm, out0_hbm, out1_hbm, acc,
                   e_v0, e_v1, si_v0, si_v1, di_v0, di_v1, sem0, sem1):
        c = lax.axis_index("c")
        s = lax.axis_index("s")
        wid = c * NS + s
        e_v = (e_v0, e_v1)
        si_v = (si_v0, si_v1)
        di_v = (di_v0, di_v1)
        sem = (sem0, sem1)

        def zrow(i, carry):
            for j in range(D // LANES):
                e_v0[i, pl.ds(j * LANES, LANES)] = jnp.zeros((LANES,), jnp.float32)
            return carry

        lax.fori_loop(0, CH, zrow, 0)

        def zcopy(i, carry):
            pltpu.sync_copy(e_v0, acc.at[pl.ds(s * z_per_tile + i * ZCH, ZCH)])
            return carry

        lax.fori_loop(0, n_z, zcopy, 0)
        plsc.subcore_barrier()

        base0 = wid * e_per_w

        def start_load(b, chunk):
            base = base0 + chunk * CH
            pltpu.async_copy(src_hbm.at[pl.ds(base, CH)], si_v[b], sem[b])
            pltpu.async_copy(dst_hbm.at[pl.ds(base, CH)], di_v[b], sem[b])
            pltpu.async_copy(e_hbm.at[pl.ds(base, CH)], e_v[b], sem[b])

        def wait_load(b):
            pltpu.make_async_copy(src_hbm.at[pl.ds(base0, CH)], si_v[b], sem[b]).wait()
            pltpu.make_async_copy(dst_hbm.at[pl.ds(base0, CH)], di_v[b], sem[b]).wait()
            pltpu.make_async_copy(e_hbm.at[pl.ds(base0, CH)], e_v[b], sem[b]).wait()

        def scatter(b):
            pltpu.sync_copy(e_v[b], acc.at[si_v[b]], add=True)
            pltpu.sync_copy(e_v[b], acc.at[di_v[b]], add=True)

        start_load(0, 0)

        n_outer = (n_ch - 1) // 2

        def step(g, carry):
            wait_load(0)
            start_load(1, 2 * g + 1)
            scatter(0)
            wait_load(1)

            @pl.when(2 * g + 2 < n_ch)
            def _():
                start_load(0, 2 * g + 2)

            scatter(1)
            return carry

        lax.fori_loop(0, n_outer, step, 0)
        if n_ch % 2 == 1:
            wait_load(0)
            scatter(0)
        plsc.subcore_barrier()

        row_slice = pl.ds(s * z_per_tile, z_per_tile)

        @pl.when(c == 0)
        def _():
            pltpu.sync_copy(acc.at[row_slice], out0_hbm.at[row_slice])

        @pl.when(c == 1)
        def _():
            pltpu.sync_copy(acc.at[row_slice], out1_hbm.at[row_slice])

    return sc_scatter


@functools.lru_cache(maxsize=None)
def _make_tc_matmul(N, D):
    BN = 2000
    assert N % BN == 0

    def mm(v_ref, p0_ref, p1_ref, w0_ref, w1_ref, b_ref, o_ref):
        v1 = p0_ref[...] + p1_ref[...]
        o_ref[...] = (
            jnp.dot(v_ref[...], w0_ref[...], preferred_element_type=jnp.float32)
            + jnp.dot(v1, w1_ref[...], preferred_element_type=jnp.float32)
            + b_ref[...]
        )

    return pl.pallas_call(
        mm,
        grid=(N // BN,),
        in_specs=[
            pl.BlockSpec((BN, D), lambda i: (i, 0)),
            pl.BlockSpec((BN, D), lambda i: (i, 0)),
            pl.BlockSpec((BN, D), lambda i: (i, 0)),
            pl.BlockSpec((D, D), lambda i: (0, 0)),
            pl.BlockSpec((D, D), lambda i: (0, 0)),
            pl.BlockSpec((1, D), lambda i: (0, 0)),
        ],
        out_specs=pl.BlockSpec((BN, D), lambda i: (i, 0)),
        out_shape=jax.ShapeDtypeStruct((N, D), jnp.float32),
    )


def kernel(e, v, edges, W, b):
    E, D = e.shape
    N = v.shape[0]
    src = edges[:, 0]
    dst = edges[:, 1]
    p0, p1 = _make_sc_scatter(N, E, D)(e, src, dst)
    mm = _make_tc_matmul(N, D)
    return mm(v, p0, p1, W[:D], W[D:], b.reshape(1, D))


# speedup vs baseline: 9.4923x; 1.0244x over previous
import functools

import jax
import jax.numpy as jnp
from jax import lax
from jax.experimental import pallas as pl
from jax.experimental.pallas import tpu as pltpu
from jax.experimental.pallas import tpu_sc as plsc

NC = 2
NS = 16
LANES = 16
NBUF = 4


@functools.lru_cache(maxsize=None)
def _make_sc_scatter(N, E, D):
    NW = NC * NS
    assert E % NW == 0, E
    e_per_w = E // NW
    CH = 80
    assert e_per_w % CH == 0, e_per_w
    n_ch = e_per_w // CH
    ZCH = CH
    n_pad = -(-N // (NS * ZCH)) * (NS * ZCH)
    z_per_tile = n_pad // NS
    n_z = z_per_tile // ZCH

    mesh = plsc.VectorSubcoreMesh(core_axis_name="c", subcore_axis_name="s")

    @functools.partial(
        pl.kernel,
        out_type=(
            jax.ShapeDtypeStruct((n_pad, D), jnp.float32),
            jax.ShapeDtypeStruct((n_pad, D), jnp.float32),
        ),
        mesh=mesh,
        scratch_types=[
            pltpu.VMEM_SHARED((n_pad, D), jnp.float32),
            [pltpu.VMEM((CH, D), jnp.float32) for _ in range(NBUF)],
            [pltpu.VMEM((CH,), jnp.int32) for _ in range(NBUF)],
            [pltpu.VMEM((CH,), jnp.int32) for _ in range(NBUF)],
            [pltpu.SemaphoreType.DMA for _ in range(NBUF)],
            [pltpu.SemaphoreType.DMA for _ in range(NBUF)],
        ],
    )
    def sc_scatter(e_hbm, src_hbm, dst_hbm, out0_hbm, out1_hbm, acc,
                   e_v, si_v, di_v, lsem, ssem):
        c = lax.axis_index("c")
        s = lax.axis_index("s")
        wid = c * NS + s
        e_v0 = e_v[0]

        def zrow(i, carry):
            for j in range(D // LANES):
                e_v0[i, pl.ds(j * LANES, LANES)] = jnp.zeros((LANES,), jnp.float32)
            return carry

        lax.fori_loop(0, CH, zrow, 0)

        def zcopy(i, carry):
            pltpu.sync_copy(e_v0, acc.at[pl.ds(s * z_per_tile + i * ZCH, ZCH)])
            return carry

        lax.fori_loop(0, n_z, zcopy, 0)
        plsc.subcore_barrier()

        base0 = wid * e_per_w

        def start_load(b, chunk):
            base = base0 + chunk * CH
            pltpu.async_copy(src_hbm.at[pl.ds(base, CH)], si_v[b], lsem[b])
            pltpu.async_copy(dst_hbm.at[pl.ds(base, CH)], di_v[b], lsem[b])
            pltpu.async_copy(e_hbm.at[pl.ds(base, CH)], e_v[b], lsem[b])

        def wait_load(b):
            pltpu.make_async_copy(src_hbm.at[pl.ds(base0, CH)], si_v[b], lsem[b]).wait()
            pltpu.make_async_copy(dst_hbm.at[pl.ds(base0, CH)], di_v[b], lsem[b]).wait()
            pltpu.make_async_copy(e_hbm.at[pl.ds(base0, CH)], e_v[b], lsem[b]).wait()

        def start_scatter(b):
            pltpu.async_copy(e_v[b], acc.at[si_v[b]], ssem[b], add=True)
            pltpu.async_copy(e_v[b], acc.at[di_v[b]], ssem[b], add=True)

        def wait_scatter(b):
            pltpu.make_async_copy(e_v[b], acc.at[si_v[b]], ssem[b]).wait()
            pltpu.make_async_copy(e_v[b], acc.at[di_v[b]], ssem[b]).wait()

        def slot(i, first, last):
            b = i % NBUF
            wait_load(b)
            start_scatter(b)
            if not first:
                wait_scatter((i - 2) % NBUF)
            if not last:
                start_load((i - 2) % NBUF, i + 2)

        start_load(0, 0)
        start_load(1, 1)
        slot(0, True, False)
        slot(1, True, False)

        n_main = (n_ch - 2 - 2) // NBUF

        def step(g, carry):
            i0 = 2 + g * NBUF
            for k in range(NBUF):
                b = (2 + k) % NBUF
                i = i0 + k
                wait_load(b)
                start_scatter(b)
                wait_scatter((b - 2) % NBUF)
                start_load((b - 2) % NBUF, i + 2)
            return carry

        lax.fori_loop(0, n_main, step, 0)
        for i in range(2 + n_main * NBUF, n_ch):
            slot(i, False, i + 2 >= n_ch)
        wait_scatter((n_ch - 2) % NBUF)
        wait_scatter((n_ch - 1) % NBUF)
        plsc.subcore_barrier()

        row_slice = pl.ds(s * z_per_tile, z_per_tile)

        @pl.when(c == 0)
        def _():
            pltpu.sync_copy(acc.at[row_slice], out0_hbm.at[row_slice])

        @pl.when(c == 1)
        def _():
            pltpu.sync_copy(acc.at[row_slice], out1_hbm.at[row_slice])

    return sc_scatter


@functools.lru_cache(maxsize=None)
def _make_tc_matmul(N, D):
    BN = 2000
    assert N % BN == 0

    def mm(v_ref, p0_ref, p1_ref, w0_ref, w1_ref, b_ref, o_ref):
        v1 = p0_ref[...] + p1_ref[...]
        o_ref[...] = (
            jnp.dot(v_ref[...], w0_ref[...], preferred_element_type=jnp.float32)
            + jnp.dot(v1, w1_ref[...], preferred_element_type=jnp.float32)
            + b_ref[...]
        )

    return pl.pallas_call(
        mm,
        grid=(N // BN,),
        in_specs=[
            pl.BlockSpec((BN, D), lambda i: (i, 0)),
            pl.BlockSpec((BN, D), lambda i: (i, 0)),
            pl.BlockSpec((BN, D), lambda i: (i, 0)),
            pl.BlockSpec((D, D), lambda i: (0, 0)),
            pl.BlockSpec((D, D), lambda i: (0, 0)),
            pl.BlockSpec((1, D), lambda i: (0, 0)),
        ],
        out_specs=pl.BlockSpec((BN, D), lambda i: (i, 0)),
        out_shape=jax.ShapeDtypeStruct((N, D), jnp.float32),
    )


def kernel(e, v, edges, W, b):
    E, D = e.shape
    N = v.shape[0]
    src = edges[:, 0]
    dst = edges[:, 1]
    p0, p1 = _make_sc_scatter(N, E, D)(e, src, dst)
    mm = _make_tc_matmul(N, D)
    return mm(v, p0, p1, W[:D], W[D:], b.reshape(1, D))


# speedup vs baseline: 9.5427x; 1.0053x over previous
import functools

import jax
import jax.numpy as jnp
from jax import lax
from jax.experimental import pallas as pl
from jax.experimental.pallas import tpu as pltpu
from jax.experimental.pallas import tpu_sc as plsc

NC = 2
NS = 16
LANES = 16
NBUF = 2


@functools.lru_cache(maxsize=None)
def _make_sc_scatter(N, E, D):
    NW = NC * NS
    assert E % NW == 0, E
    e_per_w = E // NW
    CH = 128
    n_ch = e_per_w // CH
    CT = e_per_w - n_ch * CH
    assert CT % 8 == 0, CT
    ZCH = CH
    n_pad = -(-N // (NS * ZCH)) * (NS * ZCH)
    z_per_tile = n_pad // NS
    n_z = z_per_tile // ZCH

    mesh = plsc.VectorSubcoreMesh(core_axis_name="c", subcore_axis_name="s")

    @functools.partial(
        pl.kernel,
        out_type=(
            jax.ShapeDtypeStruct((n_pad, D), jnp.float32),
            jax.ShapeDtypeStruct((n_pad, D), jnp.float32),
        ),
        mesh=mesh,
        scratch_types=[
            pltpu.VMEM_SHARED((n_pad, D), jnp.float32),
            [pltpu.VMEM((CH, D), jnp.float32) for _ in range(NBUF)],
            [pltpu.VMEM((CH,), jnp.int32) for _ in range(NBUF)],
            [pltpu.VMEM((CH,), jnp.int32) for _ in range(NBUF)],
            [pltpu.SemaphoreType.DMA for _ in range(NBUF)],
            [pltpu.SemaphoreType.DMA for _ in range(NBUF)],
            pltpu.VMEM((CT, D), jnp.float32),
            pltpu.VMEM((CT,), jnp.int32),
            pltpu.VMEM((CT,), jnp.int32),
        ],
    )
    def sc_scatter(e_hbm, src_hbm, dst_hbm, out0_hbm, out1_hbm, acc,
                   e_v, si_v, di_v, lsem, ssem, et_v, sit_v, dit_v):
        c = lax.axis_index("c")
        s = lax.axis_index("s")
        wid = c * NS + s
        e_v0 = e_v[0]

        def zrow(i, carry):
            for j in range(D // LANES):
                e_v0[i, pl.ds(j * LANES, LANES)] = jnp.zeros((LANES,), jnp.float32)
            return carry

        lax.fori_loop(0, CH, zrow, 0)

        def zcopy(i, carry):
            pltpu.sync_copy(e_v0, acc.at[pl.ds(s * z_per_tile + i * ZCH, ZCH)])
            return carry

        lax.fori_loop(0, n_z, zcopy, 0)
        plsc.subcore_barrier()

        base0 = wid * e_per_w

        def start_load(b, chunk):
            base = base0 + chunk * CH
            pltpu.async_copy(src_hbm.at[pl.ds(base, CH)], si_v[b], lsem[b])
            pltpu.async_copy(dst_hbm.at[pl.ds(base, CH)], di_v[b], lsem[b])
            pltpu.async_copy(e_hbm.at[pl.ds(base, CH)], e_v[b], lsem[b])

        def wait_load(b):
            pltpu.make_async_copy(src_hbm.at[pl.ds(base0, CH)], si_v[b], lsem[b]).wait()
            pltpu.make_async_copy(dst_hbm.at[pl.ds(base0, CH)], di_v[b], lsem[b]).wait()
            pltpu.make_async_copy(e_hbm.at[pl.ds(base0, CH)], e_v[b], lsem[b]).wait()

        def start_scatter(b):
            pltpu.async_copy(e_v[b], acc.at[si_v[b]], ssem[b], add=True)
            pltpu.async_copy(e_v[b], acc.at[di_v[b]], ssem[b], add=True)

        def wait_scatter(b):
            pltpu.make_async_copy(e_v[b], acc.at[si_v[b]], ssem[b]).wait()
            pltpu.make_async_copy(e_v[b], acc.at[di_v[b]], ssem[b]).wait()

        def slot(i, first, last):
            b = i % 2
            wait_load(b)
            start_scatter(b)
            if not first:
                wait_scatter(1 - b)
            if not last:
                start_load(1 - b, i + 1)

        start_load(0, 0)
        slot(0, True, False)

        n_main = (n_ch - 1 - 1) // 2

        def step(g, carry):
            for k in range(2):
                i = 1 + 2 * g + k
                b = (1 + k) % 2
                wait_load(b)
                start_scatter(b)
                wait_scatter(1 - b)
                start_load(1 - b, i + 1)
            return carry

        lax.fori_loop(0, n_main, step, 0)
        for i in range(1 + n_main * 2, n_ch):
            slot(i, False, i + 1 >= n_ch)
        if CT:
            tbase = base0 + n_ch * CH
            pltpu.sync_copy(src_hbm.at[pl.ds(tbase, CT)], sit_v)
            pltpu.sync_copy(dst_hbm.at[pl.ds(tbase, CT)], dit_v)
            pltpu.sync_copy(e_hbm.at[pl.ds(tbase, CT)], et_v)
        wait_scatter((n_ch - 1) % 2)
        if CT:
            pltpu.sync_copy(et_v, acc.at[sit_v], add=True)
            pltpu.sync_copy(et_v, acc.at[dit_v], add=True)
        plsc.subcore_barrier()

        row_slice = pl.ds(s * z_per_tile, z_per_tile)

        @pl.when(c == 0)
        def _():
            pltpu.sync_copy(acc.at[row_slice], out0_hbm.at[row_slice])

        @pl.when(c == 1)
        def _():
            pltpu.sync_copy(acc.at[row_slice], out1_hbm.at[row_slice])

    return sc_scatter


@functools.lru_cache(maxsize=None)
def _make_tc_matmul(N, D):
    BN = 2000
    assert N % BN == 0

    def mm(v_ref, p0_ref, p1_ref, w0_ref, w1_ref, b_ref, o_ref):
        v1 = p0_ref[...] + p1_ref[...]
        o_ref[...] = (
            jnp.dot(v_ref[...], w0_ref[...], preferred_element_type=jnp.float32)
            + jnp.dot(v1, w1_ref[...], preferred_element_type=jnp.float32)
            + b_ref[...]
        )

    return pl.pallas_call(
        mm,
        grid=(N // BN,),
        in_specs=[
            pl.BlockSpec((BN, D), lambda i: (i, 0)),
            pl.BlockSpec((BN, D), lambda i: (i, 0)),
            pl.BlockSpec((BN, D), lambda i: (i, 0)),
            pl.BlockSpec((D, D), lambda i: (0, 0)),
            pl.BlockSpec((D, D), lambda i: (0, 0)),
            pl.BlockSpec((1, D), lambda i: (0, 0)),
        ],
        out_specs=pl.BlockSpec((BN, D), lambda i: (i, 0)),
        out_shape=jax.ShapeDtypeStruct((N, D), jnp.float32),
    )


def kernel(e, v, edges, W, b):
    E, D = e.shape
    N = v.shape[0]
    src = edges[:, 0]
    dst = edges[:, 1]
    p0, p1 = _make_sc_scatter(N, E, D)(e, src, dst)
    mm = _make_tc_matmul(N, D)
    return mm(v, p0, p1, W[:D], W[D:], b.reshape(1, D))
